# baseline (speedup 1.0000x reference)
import functools

import jax
import jax.numpy as jnp
from jax import lax
from jax.experimental import pallas as pl
from jax.experimental.pallas import tpu as pltpu
from jax.experimental.pallas import tpu_sc as plsc

VOCAB = 100000
DIM = 128
PAD_IDX = 0
NORM_EPS = 1e-05

_L = 16
_NVREG = DIM // _L
_CHUNK = 128
_NBUF = 5


def _lane_sum16(v):
    lanes = lax.iota(jnp.int32, _L)
    dnums = lax.GatherDimensionNumbers(
        offset_dims=(), collapsed_slice_dims=(0,), start_index_map=(0,))
    for k in (8, 4, 2, 1):
        perm = lanes ^ jnp.int32(k)
        v = v + lax.gather(v, perm[:, None], dnums, (1,),
                           mode=lax.GatherScatterMode.PROMISE_IN_BOUNDS)
    return v


def _rsqrt16(v):
    i = lax.bitcast_convert_type(v, jnp.int32)
    i = jnp.int32(0x5F3759DF) - lax.shift_right_arithmetic(i, jnp.int32(1))
    y = lax.bitcast_convert_type(i, jnp.float32)
    half = jnp.float32(0.5) * v
    for _ in range(3 - 2):
        y = y * (jnp.float32(1.5) - half * y * y)
    return y


def _sc_body(n_per_w, idx_hbm, table_hbm, gamma_hbm, beta_hbm,
             out_hbm, mask_hbm, idx_v, rows_v, gamma_v, beta_v, mask_v,
             gsem, wsem):
    nc = plsc.get_sparse_core_info().num_cores
    wid = lax.axis_index("s") * nc + lax.axis_index("c")
    base = wid * n_per_w
    nchunks = n_per_w // _CHUNK

    pltpu.sync_copy(gamma_hbm, gamma_v)
    pltpu.sync_copy(beta_hbm, beta_v)
    pltpu.sync_copy(idx_hbm.at[pl.ds(base, n_per_w)], idx_v)

    def start_gather(c, b):
        pltpu.async_copy(
            table_hbm.at[idx_v.at[pl.ds(c * _CHUNK, _CHUNK)]],
            rows_v.at[b], gsem.at[b])

    def wait_gather(b):
        pltpu.make_async_copy(
            table_hbm.at[pl.ds(0, _CHUNK), :], rows_v.at[b], gsem.at[b]).wait()

    def wait_write(b):
        pltpu.make_async_copy(
            rows_v.at[b], out_hbm.at[pl.ds(0, _CHUNK), :], wsem.at[b]).wait()

    for c in range(_NBUF - 1):
        start_gather(c, c)

    def mask_body(j, _):
        iv = idx_v[pl.ds(j * _L, _L)]
        mask_v[pl.ds(j * _L, _L)] = jnp.where(
            iv == jnp.int32(PAD_IDX), jnp.int32(1), jnp.int32(0))
        return _

    lax.fori_loop(0, n_per_w // _L, mask_body, 0, unroll=4)
    pltpu.sync_copy(mask_v, mask_hbm.at[pl.ds(base, n_per_w)])

    gvs = [gamma_v[pl.ds(j * _L, _L)] for j in range(_NVREG)]
    bvs = [beta_v[pl.ds(j * _L, _L)] for j in range(_NVREG)]
    inv_d = jnp.float32(1.0 / DIM)
    eps = jnp.float32(NORM_EPS / DIM)

    def chunk_body(c, _):
        b = c % _NBUF
        pf = c + (_NBUF - 1)

        @pl.when(pf < nchunks)
        def _prefetch():
            pb = pf % _NBUF

            @pl.when(pf >= _NBUF)
            def _reclaim():
                wait_write(pb)

            start_gather(pf, pb)

        wait_gather(b)

        @plsc.parallel_loop(0, _CHUNK, 1, unroll=5)
        def token_body(t):
            vs = [rows_v[b, t, pl.ds(j * _L, _L)] for j in range(_NVREG)]
            sqs = [v * v for v in vs]
            while len(sqs) > 1:
                sqs = [sqs[i] + sqs[i + 1] for i in range(0, len(sqs), 2)]
            ss = list(vs)
            while len(ss) > 1:
                ss = [ss[i] + ss[i + 1] for i in range(0, len(ss), 2)]
            mean = _lane_sum16(ss[0]) * inv_d
            msq = _lane_sum16(sqs[0]) * inv_d
            a = _rsqrt16(msq - mean * mean + eps)
            for j in range(_NVREG):
                rows_v[b, t, pl.ds(j * _L, _L)] = \
                    (vs[j] - mean) * (a * gvs[j]) + bvs[j]

        pltpu.async_copy(
            rows_v.at[b], out_hbm.at[pl.ds(base + c * _CHUNK, _CHUNK), :],
            wsem.at[b])
        return _

    lax.fori_loop(0, nchunks, chunk_body, 0, unroll=False)

    for b in range(_NBUF):
        wait_write(b)


@jax.jit
def kernel(token_indices, table, gamma, beta):
    bsz, seqlen = token_indices.shape
    n = bsz * seqlen
    info = plsc.get_sparse_core_info()
    nw = info.num_cores * info.num_subcores
    n_per_w = n // nw
    assert n_per_w * nw == n and n_per_w % (_CHUNK * _NBUF) == 0

    idx_flat = token_indices.reshape(n).astype(jnp.int32)
    mesh = plsc.VectorSubcoreMesh(core_axis_name="c", subcore_axis_name="s")
    run = pl.kernel(
        functools.partial(_sc_body, n_per_w),
        mesh=mesh,
        out_type=(
            jax.ShapeDtypeStruct((n, DIM), jnp.float32),
            jax.ShapeDtypeStruct((n,), jnp.int32),
        ),
        scratch_types=[
            pltpu.VMEM((n_per_w,), jnp.int32),
            pltpu.VMEM((_NBUF, _CHUNK, DIM), jnp.float32),
            pltpu.VMEM((DIM,), jnp.float32),
            pltpu.VMEM((DIM,), jnp.float32),
            pltpu.VMEM((n_per_w,), jnp.int32),
            pltpu.SemaphoreType.DMA((_NBUF,)),
            pltpu.SemaphoreType.DMA((_NBUF,)),
        ],
    )
    out_flat, mask_flat = run(idx_flat, table, gamma, beta)
    embeds = out_flat.reshape(bsz, seqlen, DIM)
    padding_mask = mask_flat.reshape(bsz, seqlen).astype(jnp.bool_)
    return embeds, padding_mask


# speedup vs baseline: 1.2441x; 1.2441x over previous
import functools

import jax
import jax.numpy as jnp
from jax import lax
from jax.experimental import pallas as pl
from jax.experimental.pallas import tpu as pltpu
from jax.experimental.pallas import tpu_sc as plsc

VOCAB = 100000
DIM = 128
PAD_IDX = 0
NORM_EPS = 1e-05

_L = 16
_NVREG = DIM // _L
_CHUNK = 128
_NBUF = 5


def _lane_sum16(v):
    lanes = lax.iota(jnp.int32, _L)
    dnums = lax.GatherDimensionNumbers(
        offset_dims=(), collapsed_slice_dims=(0,), start_index_map=(0,))
    for k in (8, 4, 2, 1):
        perm = lanes ^ jnp.int32(k)
        v = v + lax.gather(v, perm[:, None], dnums, (1,),
                           mode=lax.GatherScatterMode.PROMISE_IN_BOUNDS)
    return v


def _rsqrt16(v):
    i = lax.bitcast_convert_type(v, jnp.int32)
    i = jnp.int32(0x5F3759DF) - lax.shift_right_arithmetic(i, jnp.int32(1))
    y = lax.bitcast_convert_type(i, jnp.float32)
    half = jnp.float32(0.5) * v
    for _ in range(3 - 2):
        y = y * (jnp.float32(1.5) - half * y * y)
    return y


def _sc_body(n_per_w, idx_hbm, table_hbm, gamma_hbm, beta_hbm,
             out_hbm, mask_hbm, idx_v, rows_v, gamma_v, beta_v, mask_v,
             gsem, wsem):
    nc = plsc.get_sparse_core_info().num_cores
    wid = lax.axis_index("s") * nc + lax.axis_index("c")
    base = wid * n_per_w
    nchunks = n_per_w // _CHUNK

    pltpu.sync_copy(gamma_hbm, gamma_v)
    pltpu.sync_copy(beta_hbm, beta_v)
    pltpu.sync_copy(idx_hbm.at[pl.ds(base, n_per_w)], idx_v)

    def start_gather(c, b):
        pltpu.async_copy(
            table_hbm.at[idx_v.at[pl.ds(c * _CHUNK, _CHUNK)]],
            rows_v.at[b], gsem.at[b])

    def wait_gather(b):
        pltpu.make_async_copy(
            table_hbm.at[pl.ds(0, _CHUNK), :], rows_v.at[b], gsem.at[b]).wait()

    def wait_write(b):
        pltpu.make_async_copy(
            rows_v.at[b], out_hbm.at[pl.ds(0, _CHUNK), :], wsem.at[b]).wait()

    for c in range(_NBUF - 1):
        start_gather(c, c)

    def mask_body(j, _):
        iv = idx_v[pl.ds(j * _L, _L)]
        mask_v[pl.ds(j * _L, _L)] = jnp.where(
            iv == jnp.int32(PAD_IDX), jnp.int32(1), jnp.int32(0))
        return _

    lax.fori_loop(0, n_per_w // _L, mask_body, 0, unroll=4)
    pltpu.sync_copy(mask_v, mask_hbm.at[pl.ds(base, n_per_w)])

    gvs = [gamma_v[pl.ds(j * _L, _L)] for j in range(_NVREG)]
    bvs = [beta_v[pl.ds(j * _L, _L)] for j in range(_NVREG)]
    inv_d = jnp.float32(1.0 / DIM)
    eps = jnp.float32(NORM_EPS / DIM)

    def chunk_body(c, _):
        b = c % _NBUF
        pf = c + (_NBUF - 1)

        @pl.when(pf < nchunks)
        def _prefetch():
            pb = pf % _NBUF

            @pl.when(pf >= _NBUF)
            def _reclaim():
                wait_write(pb)

            start_gather(pf, pb)

        wait_gather(b)

        @plsc.parallel_loop(0, _CHUNK, 1, unroll=4)
        def token_body(t):
            vs = [rows_v[b, t, pl.ds(j * _L, _L)] for j in range(_NVREG)]
            sqs = [v * v for v in vs]
            while len(sqs) > 1:
                sqs = [sqs[i] + sqs[i + 1] for i in range(0, len(sqs), 2)]
            ss = list(vs)
            while len(ss) > 1:
                ss = [ss[i] + ss[i + 1] for i in range(0, len(ss), 2)]
            mean = _lane_sum16(ss[0]) * inv_d
            msq = _lane_sum16(sqs[0]) * inv_d
            a = _rsqrt16(msq - mean * mean + eps)
            for j in range(_NVREG):
                rows_v[b, t, pl.ds(j * _L, _L)] = \
                    (vs[j] - mean) * (a * gvs[j]) + bvs[j]

        pltpu.async_copy(
            rows_v.at[b], out_hbm.at[pl.ds(base + c * _CHUNK, _CHUNK), :],
            wsem.at[b])
        return _

    lax.fori_loop(0, nchunks, chunk_body, 0, unroll=False)

    for b in range(_NBUF):
        wait_write(b)


@jax.jit
def kernel(token_indices, table, gamma, beta):
    bsz, seqlen = token_indices.shape
    n = bsz * seqlen
    info = plsc.get_sparse_core_info()
    nw = info.num_cores * info.num_subcores
    n_per_w = n // nw
    assert n_per_w * nw == n and n_per_w % (_CHUNK * _NBUF) == 0

    idx_flat = token_indices.reshape(n).astype(jnp.int32)
    mesh = plsc.VectorSubcoreMesh(core_axis_name="c", subcore_axis_name="s")
    run = pl.kernel(
        functools.partial(_sc_body, n_per_w),
        mesh=mesh,
        out_type=(
            jax.ShapeDtypeStruct((n, DIM), jnp.float32),
            jax.ShapeDtypeStruct((n,), jnp.int32),
        ),
        scratch_types=[
            pltpu.VMEM((n_per_w,), jnp.int32),
            pltpu.VMEM((_NBUF, _CHUNK, DIM), jnp.float32),
            pltpu.VMEM((DIM,), jnp.float32),
            pltpu.VMEM((DIM,), jnp.float32),
            pltpu.VMEM((n_per_w,), jnp.int32),
            pltpu.SemaphoreType.DMA((_NBUF,)),
            pltpu.SemaphoreType.DMA((_NBUF,)),
        ],
    )
    out_flat, mask_flat = run(idx_flat, table, gamma, beta)
    embeds = out_flat.reshape(bsz, seqlen, DIM)
    padding_mask = mask_flat.reshape(bsz, seqlen).astype(jnp.bool_)
    return embeds, padding_mask


# speedup vs baseline: 1.2637x; 1.0157x over previous
import functools

import jax
import jax.numpy as jnp
from jax import lax
from jax.experimental import pallas as pl
from jax.experimental.pallas import tpu as pltpu
from jax.experimental.pallas import tpu_sc as plsc

VOCAB = 100000
DIM = 128
PAD_IDX = 0
NORM_EPS = 1e-05

_L = 16
_NVREG = DIM // _L
_GATHER = 128
_CHUNK = 256
_NBUF = 3


def _lane_sum16(v):
    lanes = lax.iota(jnp.int32, _L)
    dnums = lax.GatherDimensionNumbers(
        offset_dims=(), collapsed_slice_dims=(0,), start_index_map=(0,))
    for k in (8, 4, 2, 1):
        perm = lanes ^ jnp.int32(k)
        v = v + lax.gather(v, perm[:, None], dnums, (1,),
                           mode=lax.GatherScatterMode.PROMISE_IN_BOUNDS)
    return v


def _rsqrt16(v):
    i = lax.bitcast_convert_type(v, jnp.int32)
    i = jnp.int32(0x5F3759DF) - lax.shift_right_arithmetic(i, jnp.int32(1))
    y = lax.bitcast_convert_type(i, jnp.float32)
    half = jnp.float32(0.5) * v
    for _ in range(3 - 2):
        y = y * (jnp.float32(1.5) - half * y * y)
    return y


def _sc_body(n_per_w, idx_hbm, table_hbm, gamma_hbm, beta_hbm,
             out_hbm, mask_hbm, idx_v, rows_v, gamma_v, beta_v, mask_v,
             gsem, wsem):
    nc = plsc.get_sparse_core_info().num_cores
    wid = lax.axis_index("s") * nc + lax.axis_index("c")
    base = wid * n_per_w
    nchunks = n_per_w // _CHUNK

    pltpu.sync_copy(gamma_hbm, gamma_v)
    pltpu.sync_copy(beta_hbm, beta_v)
    pltpu.sync_copy(idx_hbm.at[pl.ds(base, n_per_w)], idx_v)

    def start_gather(c, b):
        for h in range(_CHUNK // _GATHER):
            pltpu.async_copy(
                table_hbm.at[idx_v.at[pl.ds(c * _CHUNK + h * _GATHER,
                                            _GATHER)]],
                rows_v.at[b].at[pl.ds(h * _GATHER, _GATHER), :],
                gsem.at[b])

    def wait_gather(b):
        pltpu.make_async_copy(
            table_hbm.at[pl.ds(0, _CHUNK), :], rows_v.at[b], gsem.at[b]).wait()

    def wait_write(b):
        pltpu.make_async_copy(
            rows_v.at[b], out_hbm.at[pl.ds(0, _CHUNK), :], wsem.at[b]).wait()

    for c in range(_NBUF - 1):
        start_gather(c, c)

    def mask_body(j, _):
        iv = idx_v[pl.ds(j * _L, _L)]
        mask_v[pl.ds(j * _L, _L)] = jnp.where(
            iv == jnp.int32(PAD_IDX), jnp.int32(1), jnp.int32(0))
        return _

    lax.fori_loop(0, n_per_w // _L, mask_body, 0, unroll=4)
    pltpu.sync_copy(mask_v, mask_hbm.at[pl.ds(base, n_per_w)])

    gvs = [gamma_v[pl.ds(j * _L, _L)] for j in range(_NVREG)]
    bvs = [beta_v[pl.ds(j * _L, _L)] for j in range(_NVREG)]
    inv_d = jnp.float32(1.0 / DIM)
    eps = jnp.float32(NORM_EPS / DIM)

    def chunk_body(c, _):
        b = c % _NBUF
        pf = c + (_NBUF - 1)

        @pl.when(pf < nchunks)
        def _prefetch():
            pb = pf % _NBUF

            @pl.when(pf >= _NBUF)
            def _reclaim():
                wait_write(pb)

            start_gather(pf, pb)

        wait_gather(b)

        @plsc.parallel_loop(0, _CHUNK, 1, unroll=4)
        def token_body(t):
            vs = [rows_v[b, t, pl.ds(j * _L, _L)] for j in range(_NVREG)]
            sqs = [v * v for v in vs]
            while len(sqs) > 1:
                sqs = [sqs[i] + sqs[i + 1] for i in range(0, len(sqs), 2)]
            ss = list(vs)
            while len(ss) > 1:
                ss = [ss[i] + ss[i + 1] for i in range(0, len(ss), 2)]
            mean = _lane_sum16(ss[0]) * inv_d
            msq = _lane_sum16(sqs[0]) * inv_d
            a = _rsqrt16(msq - mean * mean + eps)
            for j in range(_NVREG):
                rows_v[b, t, pl.ds(j * _L, _L)] = \
                    (vs[j] - mean) * (a * gvs[j]) + bvs[j]

        pltpu.async_copy(
            rows_v.at[b], out_hbm.at[pl.ds(base + c * _CHUNK, _CHUNK), :],
            wsem.at[b])
        return _

    lax.fori_loop(0, nchunks, chunk_body, 0, unroll=False)

    for b in range(_NBUF):
        wait_write(b)


@jax.jit
def kernel(token_indices, table, gamma, beta):
    bsz, seqlen = token_indices.shape
    n = bsz * seqlen
    info = plsc.get_sparse_core_info()
    nw = info.num_cores * info.num_subcores
    n_per_w = n // nw
    assert n_per_w * nw == n and n_per_w % _CHUNK == 0
    assert n_per_w // _CHUNK >= _NBUF

    idx_flat = token_indices.reshape(n).astype(jnp.int32)
    mesh = plsc.VectorSubcoreMesh(core_axis_name="c", subcore_axis_name="s")
    run = pl.kernel(
        functools.partial(_sc_body, n_per_w),
        mesh=mesh,
        out_type=(
            jax.ShapeDtypeStruct((n, DIM), jnp.float32),
            jax.ShapeDtypeStruct((n,), jnp.int32),
        ),
        scratch_types=[
            pltpu.VMEM((n_per_w,), jnp.int32),
            pltpu.VMEM((_NBUF, _CHUNK, DIM), jnp.float32),
            pltpu.VMEM((DIM,), jnp.float32),
            pltpu.VMEM((DIM,), jnp.float32),
            pltpu.VMEM((n_per_w,), jnp.int32),
            pltpu.SemaphoreType.DMA((_NBUF,)),
            pltpu.SemaphoreType.DMA((_NBUF,)),
        ],
    )
    out_flat, mask_flat = run(idx_flat, table, gamma, beta)
    embeds = out_flat.reshape(bsz, seqlen, DIM)
    padding_mask = mask_flat.reshape(bsz, seqlen).astype(jnp.bool_)
    return embeds, padding_mask
